# baseline (speedup 1.0000x reference)
import jax
import jax.numpy as jnp
import numpy as np
from jax.experimental import pallas as pl
from jax.experimental.pallas import tpu as pltpu

_MAX_STEPS = 64
_TINY = np.float32(np.finfo(np.float32).tiny)
_NEG = np.float32(-3.0e38)
_BIGI = np.int32(1 << 30)

_N_A = 1_000_000
_R_A = 3968
_C_A = _R_A * 128
_NB_A = 2
_TR = 64

_N_T = 100_000
_R_T = 800
_C_T = _R_T * 128

_KA0 = np.array([0x3C54DD4A, 0xBDFB82F1, 0x5FB5C404, 0x92D0D25D, 0x047C950E, 0x72591913, 0x94B14B8F, 0x3D60FF5C, 0xDAE96567, 0x9D5E55EA, 0x2740747B, 0xD5873121, 0xAFA386B6, 0xF56004EB, 0x193C4E20, 0x3F7A011A, 0x6CB123AA, 0x29C8F881, 0x46473B5F, 0x21D92E84, 0x77FFD0A0, 0x19608733, 0x3FEC5D75, 0x68085B0E, 0x3496BC89, 0xC05B634F, 0xA24A14B4, 0xFF60A8BD, 0xF836C352, 0x1FE1BEDE, 0x721DE893, 0x6CFD068D, 0x46144353, 0x97672581, 0x79BE0DE2, 0x763BC343, 0xE57AC010, 0x51570C5C, 0xABBD08C6, 0x55AC63C6, 0x2C4146B0, 0x3498AFB5, 0x43D84E10, 0x8BD90F53, 0xF599C640, 0x384CBCA9, 0xC3A0AE9D, 0xED7F1253, 0x793AFCDA, 0x07D45E8A, 0x6D81BB9B, 0x6DE2E2A5, 0x442D9013, 0xF8E3FAC5, 0x75A6905A, 0x08C17082, 0x5339A6AB, 0x7B558880, 0xF62CF119, 0x2C3C8AF6, 0x929F6D1D, 0x2E919D39, 0x7227F097, 0xCA2D7F7B], dtype=np.uint32)
_KA1 = np.array([0xBBEBF007, 0x07B3B635, 0xF5658DEC, 0xF1469F5A, 0xBA192D12, 0x1855F369, 0xF2FAED4A, 0xA073FB32, 0x544A9D4C, 0x8B91B741, 0xA15D82D6, 0x510E81FA, 0x8596E76F, 0x35BA3120, 0x1FCDE381, 0x5B9697A3, 0xEFDF88E8, 0x06F1A5B4, 0x5FACA345, 0x36C16C86, 0x05F0874B, 0xDE44305D, 0x3679B90E, 0x2F8356F2, 0xC23E03CF, 0x48D82219, 0xE9307E3A, 0x37B7351F, 0x0BBFB90C, 0x1228B211, 0x11DAF6D9, 0x2EE32FCE, 0x96B4CE74, 0x6E61E6F0, 0xDD897C9B, 0xB831D75C, 0x39CC0A54, 0xEDF22A4B, 0x175A8518, 0xB1E9ADDB, 0x17B7D089, 0xF824C991, 0xF2598F3E, 0x015AF04F, 0xA7EF9267, 0xC290D38C, 0x7FA1770A, 0x63833F5C, 0x31A8453D, 0x3B94296E, 0x404A7FFB, 0x9F7E88B2, 0x43B1F965, 0x8A2457B0, 0x1AF5D969, 0xBD83CD3E, 0xB21C4789, 0x6C4BC136, 0x826BDF89, 0xE998945E, 0x459C38EF, 0xBD18D0E6, 0x89B82DA5, 0x1C212CB3], dtype=np.uint32)
_KB0 = np.array([0x65AE5E0E, 0x8C1266AC, 0xAA195163, 0x72218916, 0x698F3D8D, 0xDFC7E58C, 0xBEFE3023, 0xE94533CC, 0x29515BE5, 0x5AA670F9, 0x4EF5E775, 0xC89B4E42, 0x3FF1EFF5, 0x72FE0A2D, 0xB710F458, 0x7FBB62AF, 0xC7DD0980, 0xA7E46505, 0xD93AC099, 0x4A2E8563, 0xC4222B6C, 0x7BA9C167, 0xF0B46E84, 0xF1ACC9E7, 0x76510449, 0xE92CDF7D, 0x32361A0E, 0xC7A37FEB, 0x1CDB2494, 0xFE84577B, 0xAF58DCCC, 0xD734B217, 0x76198CE8, 0x152E098A, 0x4D5ABC9D, 0xABC5F7B3, 0x02223F5D, 0x38FB1B40, 0x4B1B8BA4, 0xC1605298, 0x15836F60, 0x9E2939E3, 0x39A22E36, 0xF0BEC4E9, 0x980727CB, 0xC7F3802A, 0x27FF7FB8, 0xC1A71620, 0xC4C318AA, 0x7B76D840, 0x6D84B81F, 0x08EA5102, 0x62171183, 0x3D7ACF67, 0x6B472F58, 0xCB41D823, 0xCF0C588D, 0x0D524532, 0x753A7887, 0xAFE5DDEB, 0xF811D876, 0x9E9DEFB1, 0xAFEBBE73, 0x93CF346C], dtype=np.uint32)
_KB1 = np.array([0x3596DFCE, 0x45A3D6BE, 0x12AA0B21, 0x67D344BE, 0xF82F75A0, 0xDA394072, 0xFC0AB783, 0x9641152F, 0x592854C1, 0xEF05ED0E, 0x4DA2156E, 0x32B37C0B, 0x2456ED85, 0x50ABB3A7, 0xC6F9D422, 0xCDA1BEDA, 0x6AA8D058, 0x380A9B77, 0x4A341201, 0x859459EA, 0x377BA180, 0x5F4585AD, 0x07FCD98B, 0x9D4C36C8, 0xFC6D1145, 0xFDB1F9A0, 0x81B9CC13, 0xA27F9538, 0x8236DDB2, 0x5FD2ACAB, 0x5D3948A9, 0x1110D9D6, 0x78EEDF47, 0x46893098, 0xD7F5E6C0, 0xF069C6DF, 0x4E536635, 0x3C2D7973, 0x543133C2, 0xFC3F0547, 0x0AAEA910, 0x3F0C3341, 0x6751983E, 0xF893DCA9, 0x90A3A523, 0xAE0B668B, 0xDAC203C2, 0xA990F30C, 0xB70DF8FC, 0xCEF9C420, 0xBEA4F259, 0x7BC38978, 0xC77DAAE7, 0x944E6FD3, 0x7C55067D, 0xAB11D265, 0x673E41EC, 0x046A2FC1, 0x89B702E0, 0x0FD3360F, 0x8877FE8C, 0x750436E5, 0x8BE8226C, 0xB62F14DD], dtype=np.uint32)
_KT0, _KT1 = np.uint32(0x00000000), np.uint32(0x00000007)


def _threefry2x32(k0, k1, x0, x1):
    ks2 = k0 ^ k1 ^ jnp.uint32(0x1BD11BDA)
    ks = (k0, k1, ks2)
    rots = ((13, 15, 26, 6), (17, 29, 16, 24))
    x0 = x0 + k0
    x1 = x1 + k1
    for i in range(5):
        for r in rots[i % 2]:
            x0 = x0 + x1
            x1 = (x1 << r) | (x1 >> (32 - r))
            x1 = x1 ^ x0
        x0 = x0 + ks[(i + 1) % 3]
        x1 = x1 + ks[(i + 2) % 3] + jnp.uint32(i + 1)
    return x0, x1


def _gumbel_from_counts(k0, k1, lo):
    o0, o1 = _threefry2x32(k0, k1, jnp.zeros_like(lo), lo)
    bits = o0 ^ o1
    u = jax.lax.bitcast_convert_type(
        (bits >> 9) | jnp.uint32(0x3F800000), jnp.float32) - 1.0
    u = jnp.maximum(_TINY, u)
    return -jnp.log(-jnp.log(u))


def _tile_iota(rows):
    return (jax.lax.broadcasted_iota(jnp.int32, (rows, 128), 0) * 128
            + jax.lax.broadcasted_iota(jnp.int32, (rows, 128), 1))


def _kernel(ka0_ref, ka1_ref, kb0_ref, kb1_ref, cs_ref, tp_ref, ap_ref,
            lp_ref, tok_ref, na_ref,
            bv_ref, bi_ref, bl_ref, ms_ref, ss_ref, j64_ref, ex_ref):
    i = pl.program_id(0)
    iota = _tile_iota(_TR)

    @pl.when(i == 0)
    def _prologue():
        kb0 = kb0_ref[...]
        kb1 = kb1_ref[...]
        z = jnp.zeros_like(kb0)
        o0, o1 = _threefry2x32(kb0, kb1, z, z)
        bits = o0 ^ o1
        u = jax.lax.bitcast_convert_type(
            (bits >> 9) | jnp.uint32(0x3F800000), jnp.float32) - 1.0
        csl = cs_ref[0, 0]
        p_change = jax.nn.sigmoid(csl)
        cs = u < p_change
        idx = jax.lax.broadcasted_iota(jnp.int32, (1, _MAX_STEPS), 1)
        first = jnp.min(jnp.where(cs, idx, _MAX_STEPS))
        j64_ref[0] = jnp.minimum(first, _MAX_STEPS - 1)
        lp1 = jax.nn.log_sigmoid(csl)
        lp0 = jax.nn.log_sigmoid(-csl)
        sum_cs = jnp.where(
            first < _MAX_STEPS,
            first.astype(jnp.float32) * lp0 + lp1,
            jnp.float32(_MAX_STEPS) * lp0)

        lt2d = tp_ref[...].reshape(_R_T, 128)
        bestv = jnp.full((_TR, 128), _NEG, jnp.float32)
        besti = jnp.zeros((_TR, 128), jnp.int32)
        bestl = jnp.full((_TR, 128), _NEG, jnp.float32)
        vm = jnp.full((_TR, 128), _NEG, jnp.float32)
        vs = jnp.zeros((_TR, 128), jnp.float32)
        for t in range(_R_T // _TR):
            et = iota + t * (_TR * 128)
            lt = jnp.where(et < _N_T, lt2d[t * _TR:(t + 1) * _TR, :], _NEG)
            g = _gumbel_from_counts(_KT0, _KT1, et.astype(jnp.uint32))
            s = lt + g
            upd = s > bestv
            bestv = jnp.where(upd, s, bestv)
            besti = jnp.where(upd, et, besti)
            bestl = jnp.where(upd, lt, bestl)
            vm2 = jnp.maximum(vm, lt)
            vs = vs * jnp.exp(vm - vm2) + jnp.exp(lt - vm2)
            vm = vm2
        m = jnp.max(bestv)
        arg = jnp.min(jnp.where(bestv == m, besti, _BIGI))
        wl = jnp.max(jnp.where(besti == arg, bestl, _NEG))
        mb = jnp.max(vm)
        lse_t = mb + jnp.log(jnp.sum(vs * jnp.exp(vm - mb)))
        na_ref[0] = arg
        ex_ref[0] = sum_cs + (wl - lse_t)

        def body(j, _):
            bv_ref[j] = _NEG
            bi_ref[j] = jnp.int32(0)
            bl_ref[j] = jnp.float32(0.0)
            return 0
        jax.lax.fori_loop(0, _MAX_STEPS, body, 0)
        ms_ref[...] = jnp.full((1, 128), _NEG, jnp.float32)
        ss_ref[...] = jnp.zeros((1, 128), jnp.float32)

    count = j64_ref[0] + 1
    base = i * _C_A
    l2d = ap_ref[...].reshape(_R_A, 128)

    vm = jnp.full((_TR, 128), _NEG, jnp.float32)
    vs = jnp.zeros((_TR, 128), jnp.float32)
    for t in range(_R_A // _TR):
        et = iota + (base + t * (_TR * 128))
        lt = jnp.where(et < _N_A, l2d[t * _TR:(t + 1) * _TR, :], _NEG)
        vm2 = jnp.maximum(vm, lt)
        vs = vs * jnp.exp(vm - vm2) + jnp.exp(lt - vm2)
        vm = vm2
    m_i = jnp.max(vm)
    s_i = jnp.sum(vs * jnp.exp(vm - m_i))
    lane = jax.lax.broadcasted_iota(jnp.int32, (1, 128), 1)
    ms_ref[...] = jnp.where(lane == i, m_i, ms_ref[...])
    ss_ref[...] = jnp.where(lane == i, s_i, ss_ref[...])

    def step(j, _):
        k0 = ka0_ref[j]
        k1 = ka1_ref[j]
        bestv = jnp.full((_TR, 128), _NEG, jnp.float32)
        besti = jnp.zeros((_TR, 128), jnp.int32)
        bestl = jnp.full((_TR, 128), _NEG, jnp.float32)
        for t in range(_R_A // _TR):
            et = iota + (base + t * (_TR * 128))
            lt = jnp.where(et < _N_A, l2d[t * _TR:(t + 1) * _TR, :], _NEG)
            g = _gumbel_from_counts(k0, k1, et.astype(jnp.uint32))
            s = lt + g
            upd = s > bestv
            bestv = jnp.where(upd, s, bestv)
            besti = jnp.where(upd, et, besti)
            bestl = jnp.where(upd, lt, bestl)
        m = jnp.max(bestv)
        arg = jnp.min(jnp.where(bestv == m, besti, _BIGI))
        wl = jnp.max(jnp.where(besti == arg, bestl, _NEG))
        bvj = bv_ref[j]
        upd = m > bvj
        bv_ref[j] = jnp.where(upd, m, bvj)
        bi_ref[j] = jnp.where(upd, arg, bi_ref[j])
        bl_ref[j] = jnp.where(upd, wl, bl_ref[j])
        return 0

    jax.lax.fori_loop(0, count, step, 0)

    @pl.when(i == _NB_A - 1)
    def _finalize():
        ms = ms_ref[...]
        gm = jnp.max(ms)
        total_s = jnp.sum(ss_ref[...] * jnp.exp(ms - gm))
        lse_a = gm + jnp.log(total_s)

        def acc(j, t):
            return t + bl_ref[j]
        sum_logit = jax.lax.fori_loop(0, count, acc, jnp.float32(0.0))
        total = (sum_logit - count.astype(jnp.float32) * lse_a + ex_ref[0])
        lp_ref[0] = total
        tok_ref[0] = bi_ref[j64_ref[0]]


@jax.jit
def kernel(action_params, transition_params, change_state):
    cs = change_state.reshape(1, 1)

    total_lp, last_tok, next_action = pl.pallas_call(
        _kernel,
        grid=(_NB_A,),
        out_shape=[
            jax.ShapeDtypeStruct((1,), jnp.float32),
            jax.ShapeDtypeStruct((1,), jnp.int32),
            jax.ShapeDtypeStruct((1,), jnp.int32),
        ],
        in_specs=[
            pl.BlockSpec(memory_space=pltpu.SMEM),
            pl.BlockSpec(memory_space=pltpu.SMEM),
            pl.BlockSpec((1, _MAX_STEPS), lambda i: (0, 0)),
            pl.BlockSpec((1, _MAX_STEPS), lambda i: (0, 0)),
            pl.BlockSpec((1, 1), lambda i: (0, 0)),
            pl.BlockSpec((_C_T,), lambda i: (0,)),
            pl.BlockSpec((_C_A,), lambda i: (i,)),
        ],
        out_specs=[
            pl.BlockSpec(memory_space=pltpu.SMEM),
            pl.BlockSpec(memory_space=pltpu.SMEM),
            pl.BlockSpec(memory_space=pltpu.SMEM),
        ],
        scratch_shapes=[
            pltpu.SMEM((_MAX_STEPS,), jnp.float32),
            pltpu.SMEM((_MAX_STEPS,), jnp.int32),
            pltpu.SMEM((_MAX_STEPS,), jnp.float32),
            pltpu.VMEM((1, 128), jnp.float32),
            pltpu.VMEM((1, 128), jnp.float32),
            pltpu.SMEM((1,), jnp.int32),
            pltpu.SMEM((1,), jnp.float32),
        ],
    )(jnp.asarray(_KA0), jnp.asarray(_KA1),
      jnp.asarray(_KB0.reshape(1, -1)), jnp.asarray(_KB1.reshape(1, -1)),
      cs, transition_params, action_params)

    return (total_lp.reshape(()), last_tok.reshape(()),
            next_action.reshape(()))


# speedup vs baseline: 1.0070x; 1.0070x over previous
import jax
import jax.numpy as jnp
import numpy as np
from jax.experimental import pallas as pl
from jax.experimental.pallas import tpu as pltpu

_MAX_STEPS = 64
_TINY = np.float32(np.finfo(np.float32).tiny)
_NEG = np.float32(-3.0e38)
_BIGI = np.int32(1 << 30)

_N_A = 1_000_000
_R_A = 3968
_C_A = _R_A * 128
_NB_A = 2
_TR = 32

_N_T = 100_000
_R_T = 800
_C_T = _R_T * 128

_KA0 = np.array([0x3C54DD4A, 0xBDFB82F1, 0x5FB5C404, 0x92D0D25D, 0x047C950E, 0x72591913, 0x94B14B8F, 0x3D60FF5C, 0xDAE96567, 0x9D5E55EA, 0x2740747B, 0xD5873121, 0xAFA386B6, 0xF56004EB, 0x193C4E20, 0x3F7A011A, 0x6CB123AA, 0x29C8F881, 0x46473B5F, 0x21D92E84, 0x77FFD0A0, 0x19608733, 0x3FEC5D75, 0x68085B0E, 0x3496BC89, 0xC05B634F, 0xA24A14B4, 0xFF60A8BD, 0xF836C352, 0x1FE1BEDE, 0x721DE893, 0x6CFD068D, 0x46144353, 0x97672581, 0x79BE0DE2, 0x763BC343, 0xE57AC010, 0x51570C5C, 0xABBD08C6, 0x55AC63C6, 0x2C4146B0, 0x3498AFB5, 0x43D84E10, 0x8BD90F53, 0xF599C640, 0x384CBCA9, 0xC3A0AE9D, 0xED7F1253, 0x793AFCDA, 0x07D45E8A, 0x6D81BB9B, 0x6DE2E2A5, 0x442D9013, 0xF8E3FAC5, 0x75A6905A, 0x08C17082, 0x5339A6AB, 0x7B558880, 0xF62CF119, 0x2C3C8AF6, 0x929F6D1D, 0x2E919D39, 0x7227F097, 0xCA2D7F7B], dtype=np.uint32)
_KA1 = np.array([0xBBEBF007, 0x07B3B635, 0xF5658DEC, 0xF1469F5A, 0xBA192D12, 0x1855F369, 0xF2FAED4A, 0xA073FB32, 0x544A9D4C, 0x8B91B741, 0xA15D82D6, 0x510E81FA, 0x8596E76F, 0x35BA3120, 0x1FCDE381, 0x5B9697A3, 0xEFDF88E8, 0x06F1A5B4, 0x5FACA345, 0x36C16C86, 0x05F0874B, 0xDE44305D, 0x3679B90E, 0x2F8356F2, 0xC23E03CF, 0x48D82219, 0xE9307E3A, 0x37B7351F, 0x0BBFB90C, 0x1228B211, 0x11DAF6D9, 0x2EE32FCE, 0x96B4CE74, 0x6E61E6F0, 0xDD897C9B, 0xB831D75C, 0x39CC0A54, 0xEDF22A4B, 0x175A8518, 0xB1E9ADDB, 0x17B7D089, 0xF824C991, 0xF2598F3E, 0x015AF04F, 0xA7EF9267, 0xC290D38C, 0x7FA1770A, 0x63833F5C, 0x31A8453D, 0x3B94296E, 0x404A7FFB, 0x9F7E88B2, 0x43B1F965, 0x8A2457B0, 0x1AF5D969, 0xBD83CD3E, 0xB21C4789, 0x6C4BC136, 0x826BDF89, 0xE998945E, 0x459C38EF, 0xBD18D0E6, 0x89B82DA5, 0x1C212CB3], dtype=np.uint32)
_KB0 = np.array([0x65AE5E0E, 0x8C1266AC, 0xAA195163, 0x72218916, 0x698F3D8D, 0xDFC7E58C, 0xBEFE3023, 0xE94533CC, 0x29515BE5, 0x5AA670F9, 0x4EF5E775, 0xC89B4E42, 0x3FF1EFF5, 0x72FE0A2D, 0xB710F458, 0x7FBB62AF, 0xC7DD0980, 0xA7E46505, 0xD93AC099, 0x4A2E8563, 0xC4222B6C, 0x7BA9C167, 0xF0B46E84, 0xF1ACC9E7, 0x76510449, 0xE92CDF7D, 0x32361A0E, 0xC7A37FEB, 0x1CDB2494, 0xFE84577B, 0xAF58DCCC, 0xD734B217, 0x76198CE8, 0x152E098A, 0x4D5ABC9D, 0xABC5F7B3, 0x02223F5D, 0x38FB1B40, 0x4B1B8BA4, 0xC1605298, 0x15836F60, 0x9E2939E3, 0x39A22E36, 0xF0BEC4E9, 0x980727CB, 0xC7F3802A, 0x27FF7FB8, 0xC1A71620, 0xC4C318AA, 0x7B76D840, 0x6D84B81F, 0x08EA5102, 0x62171183, 0x3D7ACF67, 0x6B472F58, 0xCB41D823, 0xCF0C588D, 0x0D524532, 0x753A7887, 0xAFE5DDEB, 0xF811D876, 0x9E9DEFB1, 0xAFEBBE73, 0x93CF346C], dtype=np.uint32)
_KB1 = np.array([0x3596DFCE, 0x45A3D6BE, 0x12AA0B21, 0x67D344BE, 0xF82F75A0, 0xDA394072, 0xFC0AB783, 0x9641152F, 0x592854C1, 0xEF05ED0E, 0x4DA2156E, 0x32B37C0B, 0x2456ED85, 0x50ABB3A7, 0xC6F9D422, 0xCDA1BEDA, 0x6AA8D058, 0x380A9B77, 0x4A341201, 0x859459EA, 0x377BA180, 0x5F4585AD, 0x07FCD98B, 0x9D4C36C8, 0xFC6D1145, 0xFDB1F9A0, 0x81B9CC13, 0xA27F9538, 0x8236DDB2, 0x5FD2ACAB, 0x5D3948A9, 0x1110D9D6, 0x78EEDF47, 0x46893098, 0xD7F5E6C0, 0xF069C6DF, 0x4E536635, 0x3C2D7973, 0x543133C2, 0xFC3F0547, 0x0AAEA910, 0x3F0C3341, 0x6751983E, 0xF893DCA9, 0x90A3A523, 0xAE0B668B, 0xDAC203C2, 0xA990F30C, 0xB70DF8FC, 0xCEF9C420, 0xBEA4F259, 0x7BC38978, 0xC77DAAE7, 0x944E6FD3, 0x7C55067D, 0xAB11D265, 0x673E41EC, 0x046A2FC1, 0x89B702E0, 0x0FD3360F, 0x8877FE8C, 0x750436E5, 0x8BE8226C, 0xB62F14DD], dtype=np.uint32)
_KT0, _KT1 = np.uint32(0x00000000), np.uint32(0x00000007)


def _threefry2x32(k0, k1, x0, x1):
    ks2 = k0 ^ k1 ^ jnp.uint32(0x1BD11BDA)
    ks = (k0, k1, ks2)
    rots = ((13, 15, 26, 6), (17, 29, 16, 24))
    x0 = x0 + k0
    x1 = x1 + k1
    for i in range(5):
        for r in rots[i % 2]:
            x0 = x0 + x1
            x1 = (x1 << r) | (x1 >> (32 - r))
            x1 = x1 ^ x0
        x0 = x0 + ks[(i + 1) % 3]
        x1 = x1 + ks[(i + 2) % 3] + jnp.uint32(i + 1)
    return x0, x1


def _gumbel_from_counts(k0, k1, lo):
    o0, o1 = _threefry2x32(k0, k1, jnp.zeros_like(lo), lo)
    bits = o0 ^ o1
    u = jax.lax.bitcast_convert_type(
        (bits >> 9) | jnp.uint32(0x3F800000), jnp.float32) - 1.0
    u = jnp.maximum(_TINY, u)
    return -jnp.log(-jnp.log(u))


def _tile_iota(rows):
    return (jax.lax.broadcasted_iota(jnp.int32, (rows, 128), 0) * 128
            + jax.lax.broadcasted_iota(jnp.int32, (rows, 128), 1))


def _kernel(ka0_ref, ka1_ref, kb0_ref, kb1_ref, cs_ref, tp_ref, ap_ref,
            lp_ref, tok_ref, na_ref,
            bv_ref, bi_ref, bl_ref, ms_ref, ss_ref, j64_ref, ex_ref):
    i = pl.program_id(0)
    iota = _tile_iota(_TR)

    @pl.when(i == 0)
    def _prologue():
        kb0 = kb0_ref[...]
        kb1 = kb1_ref[...]
        z = jnp.zeros_like(kb0)
        o0, o1 = _threefry2x32(kb0, kb1, z, z)
        bits = o0 ^ o1
        u = jax.lax.bitcast_convert_type(
            (bits >> 9) | jnp.uint32(0x3F800000), jnp.float32) - 1.0
        csl = cs_ref[0, 0]
        p_change = jax.nn.sigmoid(csl)
        cs = u < p_change
        idx = jax.lax.broadcasted_iota(jnp.int32, (1, _MAX_STEPS), 1)
        first = jnp.min(jnp.where(cs, idx, _MAX_STEPS))
        j64_ref[0] = jnp.minimum(first, _MAX_STEPS - 1)
        lp1 = jax.nn.log_sigmoid(csl)
        lp0 = jax.nn.log_sigmoid(-csl)
        sum_cs = jnp.where(
            first < _MAX_STEPS,
            first.astype(jnp.float32) * lp0 + lp1,
            jnp.float32(_MAX_STEPS) * lp0)

        lt2d = tp_ref[...].reshape(_R_T, 128)
        bestv = jnp.full((_TR, 128), _NEG, jnp.float32)
        besti = jnp.zeros((_TR, 128), jnp.int32)
        bestl = jnp.full((_TR, 128), _NEG, jnp.float32)
        vm = jnp.full((_TR, 128), _NEG, jnp.float32)
        vs = jnp.zeros((_TR, 128), jnp.float32)
        for t in range(_R_T // _TR):
            et = iota + t * (_TR * 128)
            lt = jnp.where(et < _N_T, lt2d[t * _TR:(t + 1) * _TR, :], _NEG)
            g = _gumbel_from_counts(_KT0, _KT1, et.astype(jnp.uint32))
            s = lt + g
            upd = s > bestv
            bestv = jnp.where(upd, s, bestv)
            besti = jnp.where(upd, et, besti)
            bestl = jnp.where(upd, lt, bestl)
            vm2 = jnp.maximum(vm, lt)
            vs = vs * jnp.exp(vm - vm2) + jnp.exp(lt - vm2)
            vm = vm2
        m = jnp.max(bestv)
        arg = jnp.min(jnp.where(bestv == m, besti, _BIGI))
        wl = jnp.max(jnp.where(besti == arg, bestl, _NEG))
        mb = jnp.max(vm)
        lse_t = mb + jnp.log(jnp.sum(vs * jnp.exp(vm - mb)))
        na_ref[0] = arg
        ex_ref[0] = sum_cs + (wl - lse_t)

        def body(j, _):
            bv_ref[j] = _NEG
            bi_ref[j] = jnp.int32(0)
            bl_ref[j] = jnp.float32(0.0)
            return 0
        jax.lax.fori_loop(0, _MAX_STEPS, body, 0)
        ms_ref[...] = jnp.full((1, 128), _NEG, jnp.float32)
        ss_ref[...] = jnp.zeros((1, 128), jnp.float32)

    count = j64_ref[0] + 1
    base = i * _C_A
    l2d = ap_ref[...].reshape(_R_A, 128)

    vm = jnp.full((_TR, 128), _NEG, jnp.float32)
    vs = jnp.zeros((_TR, 128), jnp.float32)
    for t in range(_R_A // _TR):
        et = iota + (base + t * (_TR * 128))
        lt = jnp.where(et < _N_A, l2d[t * _TR:(t + 1) * _TR, :], _NEG)
        vm2 = jnp.maximum(vm, lt)
        vs = vs * jnp.exp(vm - vm2) + jnp.exp(lt - vm2)
        vm = vm2
    m_i = jnp.max(vm)
    s_i = jnp.sum(vs * jnp.exp(vm - m_i))
    lane = jax.lax.broadcasted_iota(jnp.int32, (1, 128), 1)
    ms_ref[...] = jnp.where(lane == i, m_i, ms_ref[...])
    ss_ref[...] = jnp.where(lane == i, s_i, ss_ref[...])

    def step(j, _):
        k0 = ka0_ref[j]
        k1 = ka1_ref[j]
        bestv = jnp.full((_TR, 128), _NEG, jnp.float32)
        besti = jnp.zeros((_TR, 128), jnp.int32)
        bestl = jnp.full((_TR, 128), _NEG, jnp.float32)
        for t in range(_R_A // _TR):
            et = iota + (base + t * (_TR * 128))
            lt = jnp.where(et < _N_A, l2d[t * _TR:(t + 1) * _TR, :], _NEG)
            g = _gumbel_from_counts(k0, k1, et.astype(jnp.uint32))
            s = lt + g
            upd = s > bestv
            bestv = jnp.where(upd, s, bestv)
            besti = jnp.where(upd, et, besti)
            bestl = jnp.where(upd, lt, bestl)
        m = jnp.max(bestv)
        arg = jnp.min(jnp.where(bestv == m, besti, _BIGI))
        wl = jnp.max(jnp.where(besti == arg, bestl, _NEG))
        bvj = bv_ref[j]
        upd = m > bvj
        bv_ref[j] = jnp.where(upd, m, bvj)
        bi_ref[j] = jnp.where(upd, arg, bi_ref[j])
        bl_ref[j] = jnp.where(upd, wl, bl_ref[j])
        return 0

    jax.lax.fori_loop(0, count, step, 0)

    @pl.when(i == _NB_A - 1)
    def _finalize():
        ms = ms_ref[...]
        gm = jnp.max(ms)
        total_s = jnp.sum(ss_ref[...] * jnp.exp(ms - gm))
        lse_a = gm + jnp.log(total_s)

        def acc(j, t):
            return t + bl_ref[j]
        sum_logit = jax.lax.fori_loop(0, count, acc, jnp.float32(0.0))
        total = (sum_logit - count.astype(jnp.float32) * lse_a + ex_ref[0])
        lp_ref[0] = total
        tok_ref[0] = bi_ref[j64_ref[0]]


@jax.jit
def kernel(action_params, transition_params, change_state):
    cs = change_state.reshape(1, 1)

    total_lp, last_tok, next_action = pl.pallas_call(
        _kernel,
        grid=(_NB_A,),
        out_shape=[
            jax.ShapeDtypeStruct((1,), jnp.float32),
            jax.ShapeDtypeStruct((1,), jnp.int32),
            jax.ShapeDtypeStruct((1,), jnp.int32),
        ],
        in_specs=[
            pl.BlockSpec(memory_space=pltpu.SMEM),
            pl.BlockSpec(memory_space=pltpu.SMEM),
            pl.BlockSpec((1, _MAX_STEPS), lambda i: (0, 0)),
            pl.BlockSpec((1, _MAX_STEPS), lambda i: (0, 0)),
            pl.BlockSpec((1, 1), lambda i: (0, 0)),
            pl.BlockSpec((_C_T,), lambda i: (0,)),
            pl.BlockSpec((_C_A,), lambda i: (i,)),
        ],
        out_specs=[
            pl.BlockSpec(memory_space=pltpu.SMEM),
            pl.BlockSpec(memory_space=pltpu.SMEM),
            pl.BlockSpec(memory_space=pltpu.SMEM),
        ],
        scratch_shapes=[
            pltpu.SMEM((_MAX_STEPS,), jnp.float32),
            pltpu.SMEM((_MAX_STEPS,), jnp.int32),
            pltpu.SMEM((_MAX_STEPS,), jnp.float32),
            pltpu.VMEM((1, 128), jnp.float32),
            pltpu.VMEM((1, 128), jnp.float32),
            pltpu.SMEM((1,), jnp.int32),
            pltpu.SMEM((1,), jnp.float32),
        ],
    )(jnp.asarray(_KA0), jnp.asarray(_KA1),
      jnp.asarray(_KB0.reshape(1, -1)), jnp.asarray(_KB1.reshape(1, -1)),
      cs, transition_params, action_params)

    return (total_lp.reshape(()), last_tok.reshape(()),
            next_action.reshape(()))
